# baseline (speedup 1.0000x reference)
import jax
import jax.numpy as jnp
from jax.experimental import pallas as pl
from jax.experimental.pallas import tpu as pltpu

B = 8
D = 512
ML = 256
T = 2 * ML - 1


def _splitnet_kernel(x_ref, wt_ref, lnw_ref, lnb_ref, lb_ref, slv_ref,
                     sls_ref, lab_ref,
                     leaf_ref, sc_ref,
                     tree_ref, stack_ref, ga_ref, ra_ref, la_ref, sf_ref,
                     scal_ref):
    leaf_ref[...] = jnp.zeros((B, ML, D), jnp.float32)
    tree_ref[...] = jnp.zeros((B, ML, D), jnp.float32)

    for b in range(B):
        scal_ref[0, b] = 0
        scal_ref[1, b] = 0
        scal_ref[2, b] = 0
        scal_ref[3, b] = 1

    def int_step(i, _):
        for b in range(B):
            act = i < sls_ref[0, b]
            sp_b = scal_ref[0, b]
            rc_b = scal_ref[1, b]
            lc_b = scal_ref[2, b]
            ps_b = scal_ref[3, b]
            need_pop = jnp.logical_and(act, jnp.logical_and(ps_b == 0, sp_b > 0))
            ga_ref[b, i] = jnp.where(need_pop, stack_ref[b, jnp.where(need_pop, sp_b - 1, 0)], 0)
            sp_b = jnp.where(need_pop, sp_b - 1, sp_b)
            split = jnp.logical_and(act, lab_ref[b, i] > 0)
            sf_ref[b, i] = jnp.where(split, 1, 0)
            stack_ref[b, jnp.where(split, sp_b, 0)] = jnp.where(split, rc_b, stack_ref[b, jnp.where(split, sp_b, 0)])
            ra_ref[b, i] = rc_b
            scal_ref[0, b] = jnp.where(split, sp_b + 1, sp_b)
            scal_ref[1, b] = jnp.where(split, rc_b + 1, rc_b)
            is_leaf = jnp.logical_and(act, jnp.logical_not(split))
            la_ref[b, i] = lc_b
            scal_ref[2, b] = jnp.where(is_leaf, lc_b + 1, lc_b)
            scal_ref[3, b] = jnp.where(act, jnp.where(split, 1, 0), ps_b)
        return 0

    steps = sls_ref[0, 0]
    for b in range(1, B):
        steps = jnp.maximum(steps, sls_ref[0, b])
    jax.lax.fori_loop(0, steps, int_step, 0)

    lnw = lnw_ref[...]
    lnb = lnb_ref[...]
    lb = lb_ref[...]
    inv_d = 1.0 / D

    def flt_step(i, carry):
        prev_left, sfv, scores = carry
        parts = []
        for b in range(B):
            parts.append(tree_ref[b, pl.ds(ga_ref[b, i], 1), :])
        gath = jnp.concatenate(parts, axis=0)
        parent = jnp.where(sfv > 0.5, prev_left, gath)

        s1 = jnp.sum(parent, axis=1, keepdims=True)
        s2 = jnp.sum(parent * parent, axis=1, keepdims=True)
        mu = s1 * inv_d
        var = s2 * inv_d - mu * mu
        xn = (parent - mu) * jax.lax.rsqrt(var + 1e-5) * lnw + lnb
        y = jnp.dot(xn, wt_ref[...], preferred_element_type=jnp.float32)
        gate = jax.nn.sigmoid(y + lb)
        left = gate * parent
        right = (1.0 - gate) * parent

        num = jnp.sum(left * right, axis=1, keepdims=True)
        na = jnp.maximum(jnp.sqrt(jnp.sum(left * left, axis=1, keepdims=True)), 1e-8)
        nb = jnp.maximum(jnp.sqrt(jnp.sum(right * right, axis=1, keepdims=True)), 1e-8)
        s = num / (na * nb)
        act_v = slv_ref[...] > i
        col = jax.lax.broadcasted_iota(jnp.int32, (B, D), 1)
        scores = scores + jnp.where((col == i) & act_v, s, 0.0)

        sf_parts = []
        for b in range(B):
            act = i < sls_ref[0, b]
            split = sf_ref[b, i] > 0

            @pl.when(split)
            def _(b=b):
                tree_ref[b, pl.ds(ra_ref[b, i], 1), :] = right[b:b + 1, :]

            @pl.when(jnp.logical_and(act, jnp.logical_not(split)))
            def _(b=b):
                leaf_ref[b, pl.ds(la_ref[b, i], 1), :] = parent[b:b + 1, :]

            sf_parts.append(jnp.full((1, 1), jnp.where(split, 1.0, 0.0), jnp.float32))
        sfv_next = jnp.concatenate(sf_parts, axis=0)
        return left, sfv_next, scores

    scores = jax.lax.fori_loop(
        0, steps, flt_step,
        (x_ref[...], jnp.ones((B, 1), jnp.float32), jnp.zeros((B, D), jnp.float32)),
    )[2]
    sc_ref[...] = scores


def kernel(input_, features, length, label, ln_weight, ln_bias, lin_weight, lin_bias):
    del features
    length = length.astype(jnp.int32)
    label = label.astype(jnp.int32)
    sl = 2 * length - 1

    leaf, scores = pl.pallas_call(
        _splitnet_kernel,
        out_shape=[
            jax.ShapeDtypeStruct((B, ML, D), jnp.float32),
            jax.ShapeDtypeStruct((B, D), jnp.float32),
        ],
        in_specs=[
            pl.BlockSpec(memory_space=pltpu.VMEM),
            pl.BlockSpec(memory_space=pltpu.VMEM),
            pl.BlockSpec(memory_space=pltpu.VMEM),
            pl.BlockSpec(memory_space=pltpu.VMEM),
            pl.BlockSpec(memory_space=pltpu.VMEM),
            pl.BlockSpec(memory_space=pltpu.VMEM),
            pl.BlockSpec(memory_space=pltpu.SMEM),
            pl.BlockSpec(memory_space=pltpu.SMEM),
        ],
        out_specs=[
            pl.BlockSpec(memory_space=pltpu.VMEM),
            pl.BlockSpec(memory_space=pltpu.VMEM),
        ],
        scratch_shapes=[
            pltpu.VMEM((B, ML, D), jnp.float32),
            pltpu.SMEM((B, D), jnp.int32),
            pltpu.SMEM((B, D), jnp.int32),
            pltpu.SMEM((B, D), jnp.int32),
            pltpu.SMEM((B, D), jnp.int32),
            pltpu.SMEM((B, D), jnp.int32),
            pltpu.SMEM((4, B), jnp.int32),
        ],
    )(
        input_,
        lin_weight.T,
        ln_weight.reshape(1, D),
        ln_bias.reshape(1, D),
        lin_bias.reshape(1, D),
        sl.reshape(B, 1),
        sl.reshape(1, B),
        label[:, :T],
    )
    return leaf, scores[:, :T]


# speedup vs baseline: 1.0883x; 1.0883x over previous
import jax
import jax.numpy as jnp
from jax.experimental import pallas as pl
from jax.experimental.pallas import tpu as pltpu

B = 8
D = 512
ML = 256
T = 2 * ML - 1


def _splitnet_kernel(x_ref, wt_ref, lnw_ref, lnb_ref, lb_ref, slv_ref,
                     sls_ref, lab_ref,
                     leaf_ref, sc_ref,
                     tree_ref, stack_ref, ga_ref, ra_ref, la_ref, sf_ref,
                     scal_ref):
    leaf_ref[...] = jnp.zeros((B, ML, D), jnp.float32)
    tree_ref[...] = jnp.zeros((B, ML, D), jnp.float32)

    for b in range(B):
        scal_ref[0, b] = 0
        scal_ref[1, b] = 0
        scal_ref[2, b] = 0
        scal_ref[3, b] = 1

    def int_step(i, _):
        for b in range(B):
            act = i < sls_ref[0, b]
            sp_b = scal_ref[0, b]
            rc_b = scal_ref[1, b]
            lc_b = scal_ref[2, b]
            ps_b = scal_ref[3, b]
            need_pop = jnp.logical_and(act, jnp.logical_and(ps_b == 0, sp_b > 0))
            ga_ref[b, i] = jnp.where(need_pop, stack_ref[b, jnp.where(need_pop, sp_b - 1, 0)], 0)
            sp_b = jnp.where(need_pop, sp_b - 1, sp_b)
            split = jnp.logical_and(act, lab_ref[b, i] > 0)
            sf_ref[b, i] = jnp.where(split, 1, 0)
            slot = jnp.where(split, sp_b, 0)
            stack_ref[b, slot] = jnp.where(split, rc_b, stack_ref[b, slot])
            ra_ref[b, i] = rc_b
            scal_ref[0, b] = jnp.where(split, sp_b + 1, sp_b)
            scal_ref[1, b] = jnp.where(split, rc_b + 1, rc_b)
            is_leaf = jnp.logical_and(act, jnp.logical_not(split))
            la_ref[b, i] = lc_b
            scal_ref[2, b] = jnp.where(is_leaf, lc_b + 1, lc_b)
            scal_ref[3, b] = jnp.where(act, jnp.where(split, 1, 0), ps_b)
        return 0

    steps = sls_ref[0, 0]
    for b in range(1, B):
        steps = jnp.maximum(steps, sls_ref[0, b])
    jax.lax.fori_loop(0, steps, int_step, 0)

    lnw = lnw_ref[...]
    lnb = lnb_ref[...]
    lb = lb_ref[...]
    inv_d = 1.0 / D
    slv = slv_ref[...]
    col = jax.lax.broadcasted_iota(jnp.int32, (B, D), 1)

    def flt_step(i, carry):
        parent_prev, gate_prev, sfv, s2_prev, scores = carry
        left_prev = gate_prev * parent_prev

        ii = jnp.minimum(i, steps - 1)
        parts = []
        for b in range(B):
            parts.append(tree_ref[b, pl.ds(ga_ref[b, ii], 1), :])
        gath = jnp.concatenate(parts, axis=0)
        parent = jnp.where(sfv > 0.5, left_prev, gath)

        s1 = jnp.sum(parent, axis=1, keepdims=True)
        s2 = jnp.sum(parent * parent, axis=1, keepdims=True)
        mu = s1 * inv_d
        var = s2 * inv_d - mu * mu
        xn = (parent - mu) * jax.lax.rsqrt(var + 1e-5) * lnw + lnb
        y = jnp.dot(xn, wt_ref[...], preferred_element_type=jnp.float32)

        im1 = jnp.maximum(i - 1, 0)
        right_prev = parent_prev - left_prev
        spq = jnp.sum(parent_prev * left_prev, axis=1, keepdims=True)
        sq2 = jnp.sum(left_prev * left_prev, axis=1, keepdims=True)
        num = spq - sq2
        na = jnp.maximum(jnp.sqrt(sq2), 1e-8)
        nb = jnp.maximum(jnp.sqrt(s2_prev - 2.0 * spq + sq2), 1e-8)
        s = num / (na * nb)
        scores = scores + jnp.where((col == i - 1) & (slv > i - 1), s, 0.0)

        sf_parts = []
        for b in range(B):
            split_prev = jnp.logical_and(i > 0, sf_ref[b, im1] > 0)
            leaf_prev = jnp.logical_and(
                i > 0,
                jnp.logical_and(im1 < sls_ref[0, b], sf_ref[b, im1] == 0))

            @pl.when(split_prev)
            def _(b=b):
                tree_ref[b, pl.ds(ra_ref[b, im1], 1), :] = right_prev[b:b + 1, :]

            @pl.when(leaf_prev)
            def _(b=b):
                leaf_ref[b, pl.ds(la_ref[b, im1], 1), :] = parent_prev[b:b + 1, :]

            sf_parts.append(jnp.full((1, 1),
                                     jnp.where(sf_ref[b, ii] > 0, 1.0, 0.0),
                                     jnp.float32))
        sfv_next = jnp.concatenate(sf_parts, axis=0)

        gate = jax.nn.sigmoid(y + lb)
        return parent, gate, sfv_next, s2, scores

    scores = jax.lax.fori_loop(
        0, steps + 1, flt_step,
        (x_ref[...], jnp.ones((B, D), jnp.float32),
         jnp.ones((B, 1), jnp.float32), jnp.zeros((B, 1), jnp.float32),
         jnp.zeros((B, D), jnp.float32)),
    )[4]
    sc_ref[...] = scores


def kernel(input_, features, length, label, ln_weight, ln_bias, lin_weight, lin_bias):
    del features
    length = length.astype(jnp.int32)
    label = label.astype(jnp.int32)
    sl = 2 * length - 1

    leaf, scores = pl.pallas_call(
        _splitnet_kernel,
        out_shape=[
            jax.ShapeDtypeStruct((B, ML, D), jnp.float32),
            jax.ShapeDtypeStruct((B, D), jnp.float32),
        ],
        in_specs=[
            pl.BlockSpec(memory_space=pltpu.VMEM),
            pl.BlockSpec(memory_space=pltpu.VMEM),
            pl.BlockSpec(memory_space=pltpu.VMEM),
            pl.BlockSpec(memory_space=pltpu.VMEM),
            pl.BlockSpec(memory_space=pltpu.VMEM),
            pl.BlockSpec(memory_space=pltpu.VMEM),
            pl.BlockSpec(memory_space=pltpu.SMEM),
            pl.BlockSpec(memory_space=pltpu.SMEM),
        ],
        out_specs=[
            pl.BlockSpec(memory_space=pltpu.VMEM),
            pl.BlockSpec(memory_space=pltpu.VMEM),
        ],
        scratch_shapes=[
            pltpu.VMEM((B, ML, D), jnp.float32),
            pltpu.SMEM((B, D), jnp.int32),
            pltpu.SMEM((B, D), jnp.int32),
            pltpu.SMEM((B, D), jnp.int32),
            pltpu.SMEM((B, D), jnp.int32),
            pltpu.SMEM((B, D), jnp.int32),
            pltpu.SMEM((4, B), jnp.int32),
        ],
    )(
        input_,
        lin_weight.T,
        ln_weight.reshape(1, D),
        ln_bias.reshape(1, D),
        lin_bias.reshape(1, D),
        sl.reshape(B, 1),
        sl.reshape(1, B),
        label[:, :T],
    )
    return leaf, scores[:, :T]


# speedup vs baseline: 1.4022x; 1.2885x over previous
import jax
import jax.numpy as jnp
from jax.experimental import pallas as pl
from jax.experimental.pallas import tpu as pltpu

B = 8
D = 512
ML = 256
T = 2 * ML - 1


def _splitnet_kernel(x_ref, wl_ref, wsum_ref, bias0_ref, slv_ref,
                     sls_ref, lab_ref,
                     leaf_ref, sc_ref,
                     tree_ref, stack_ref, ga_ref, ra_ref, la_ref, sf_ref,
                     scal_ref):
    leaf_ref[...] = jnp.zeros((B, ML, D), jnp.float32)
    tree_ref[...] = jnp.zeros((B, ML, D), jnp.float32)

    for b in range(B):
        scal_ref[0, b] = 0
        scal_ref[1, b] = 0
        scal_ref[2, b] = 0
        scal_ref[3, b] = 1

    def int_step(i, _):
        for b in range(B):
            act = i < sls_ref[0, b]
            sp_b = scal_ref[0, b]
            rc_b = scal_ref[1, b]
            lc_b = scal_ref[2, b]
            ps_b = scal_ref[3, b]
            need_pop = jnp.logical_and(act, jnp.logical_and(ps_b == 0, sp_b > 0))
            ga_ref[b, i] = jnp.where(need_pop, stack_ref[b, jnp.where(need_pop, sp_b - 1, 0)], 0)
            sp_b = jnp.where(need_pop, sp_b - 1, sp_b)
            split = jnp.logical_and(act, lab_ref[b, i] > 0)
            sf_ref[b, i] = jnp.where(split, 1, 0)
            slot = jnp.where(split, sp_b, 0)
            stack_ref[b, slot] = jnp.where(split, rc_b, stack_ref[b, slot])
            ra_ref[b, i] = rc_b
            scal_ref[0, b] = jnp.where(split, sp_b + 1, sp_b)
            scal_ref[1, b] = jnp.where(split, rc_b + 1, rc_b)
            is_leaf = jnp.logical_and(act, jnp.logical_not(split))
            la_ref[b, i] = lc_b
            scal_ref[2, b] = jnp.where(is_leaf, lc_b + 1, lc_b)
            scal_ref[3, b] = jnp.where(act, jnp.where(split, 1, 0), ps_b)
        return 0

    steps = sls_ref[0, 0]
    for b in range(1, B):
        steps = jnp.maximum(steps, sls_ref[0, b])
    jax.lax.fori_loop(0, steps, int_step, 0)

    wsum = wsum_ref[...]
    bias0 = bias0_ref[...]
    inv_d = 1.0 / D
    slv = slv_ref[...]
    col = jax.lax.broadcasted_iota(jnp.int32, (B, D), 1)

    def flt_step(i, carry):
        parent_prev, gate_prev, sfv, s2_prev, scores = carry
        left_prev = gate_prev * parent_prev

        ii = jnp.minimum(i, steps - 1)
        parts = []
        for b in range(B):
            parts.append(tree_ref[b, pl.ds(ga_ref[b, ii], 1), :])
        gath = jnp.concatenate(parts, axis=0)
        parent = jnp.where(sfv > 0.5, left_prev, gath)

        m = jnp.dot(parent, wl_ref[...], preferred_element_type=jnp.float32)
        s1 = jnp.sum(parent, axis=1, keepdims=True)
        s2 = jnp.sum(parent * parent, axis=1, keepdims=True)
        mu = s1 * inv_d
        var = s2 * inv_d - mu * mu
        istd = jax.lax.rsqrt(var + 1e-5)

        im1 = jnp.maximum(i - 1, 0)
        right_prev = parent_prev - left_prev
        spq = jnp.sum(parent_prev * left_prev, axis=1, keepdims=True)
        sq2 = jnp.sum(left_prev * left_prev, axis=1, keepdims=True)
        num = spq - sq2
        na = jnp.maximum(jnp.sqrt(sq2), 1e-8)
        nb = jnp.maximum(jnp.sqrt(s2_prev - 2.0 * spq + sq2), 1e-8)
        s = num / (na * nb)
        scores = scores + jnp.where((col == i - 1) & (slv > i - 1), s, 0.0)

        sf_parts = []
        for b in range(B):
            split_prev = jnp.logical_and(i > 0, sf_ref[b, im1] > 0)
            leaf_prev = jnp.logical_and(
                i > 0,
                jnp.logical_and(im1 < sls_ref[0, b], sf_ref[b, im1] == 0))

            @pl.when(split_prev)
            def _(b=b):
                tree_ref[b, pl.ds(ra_ref[b, im1], 1), :] = right_prev[b:b + 1, :]

            @pl.when(leaf_prev)
            def _(b=b):
                leaf_ref[b, pl.ds(la_ref[b, im1], 1), :] = parent_prev[b:b + 1, :]

            sf_parts.append(jnp.full((1, 1),
                                     jnp.where(sf_ref[b, ii] > 0, 1.0, 0.0),
                                     jnp.float32))
        sfv_next = jnp.concatenate(sf_parts, axis=0)

        gate = jax.nn.sigmoid(istd * (m - mu * wsum) + bias0)
        return parent, gate, sfv_next, s2, scores

    scores = jax.lax.fori_loop(
        0, steps + 1, flt_step,
        (x_ref[...], jnp.ones((B, D), jnp.float32),
         jnp.ones((B, 1), jnp.float32), jnp.zeros((B, 1), jnp.float32),
         jnp.zeros((B, D), jnp.float32)),
    )[4]
    sc_ref[...] = scores


def kernel(input_, features, length, label, ln_weight, ln_bias, lin_weight, lin_bias):
    del features
    length = length.astype(jnp.int32)
    label = label.astype(jnp.int32)
    sl = 2 * length - 1

    leaf, scores = pl.pallas_call(
        _splitnet_kernel,
        out_shape=[
            jax.ShapeDtypeStruct((B, ML, D), jnp.float32),
            jax.ShapeDtypeStruct((B, D), jnp.float32),
        ],
        in_specs=[
            pl.BlockSpec(memory_space=pltpu.VMEM),
            pl.BlockSpec(memory_space=pltpu.VMEM),
            pl.BlockSpec(memory_space=pltpu.VMEM),
            pl.BlockSpec(memory_space=pltpu.VMEM),
            pl.BlockSpec(memory_space=pltpu.VMEM),
            pl.BlockSpec(memory_space=pltpu.SMEM),
            pl.BlockSpec(memory_space=pltpu.SMEM),
        ],
        out_specs=[
            pl.BlockSpec(memory_space=pltpu.VMEM),
            pl.BlockSpec(memory_space=pltpu.VMEM),
        ],
        scratch_shapes=[
            pltpu.VMEM((B, ML, D), jnp.float32),
            pltpu.SMEM((B, D), jnp.int32),
            pltpu.SMEM((B, D), jnp.int32),
            pltpu.SMEM((B, D), jnp.int32),
            pltpu.SMEM((B, D), jnp.int32),
            pltpu.SMEM((B, D), jnp.int32),
            pltpu.SMEM((4, B), jnp.int32),
        ],
    )(
        input_,
        ln_weight[:, None] * lin_weight.T,
        (ln_weight @ lin_weight.T).reshape(1, D),
        (ln_bias @ lin_weight.T + lin_bias).reshape(1, D),
        sl.reshape(B, 1),
        sl.reshape(1, B),
        label[:, :T],
    )
    return leaf, scores[:, :T]


# speedup vs baseline: 1.7506x; 1.2485x over previous
import jax
import jax.numpy as jnp
from jax.experimental import pallas as pl
from jax.experimental.pallas import tpu as pltpu

B = 8
D = 512
ML = 256
T = 2 * ML - 1


def _splitnet_kernel(x_ref, wl_ref, wsum_ref, bias0_ref, slv_ref,
                     sls_ref, lab_ref,
                     leaf_ref, sc_ref,
                     tree_ref, stack_ref, ga_ref, ra_ref, la_ref, sf_ref,
                     scal_ref, lscr_ref):
    lscr_ref[...] = jnp.zeros((B, ML + 8, D), jnp.float32)
    tree_ref[...] = jnp.zeros((B, ML, D), jnp.float32)

    for b in range(B):
        scal_ref[0, b] = 0
        scal_ref[1, b] = 0
        scal_ref[2, b] = 0
        scal_ref[3, b] = 1

    def int_step(i, _):
        for b in range(B):
            act = i < sls_ref[0, b]
            sp_b = scal_ref[0, b]
            rc_b = scal_ref[1, b]
            lc_b = scal_ref[2, b]
            ps_b = scal_ref[3, b]
            need_pop = jnp.logical_and(act, jnp.logical_and(ps_b == 0, sp_b > 0))
            ga_ref[b, i] = jnp.where(need_pop, stack_ref[b, jnp.where(need_pop, sp_b - 1, 0)], 0)
            sp_b = jnp.where(need_pop, sp_b - 1, sp_b)
            split = jnp.logical_and(act, lab_ref[b, i] > 0)
            sf_ref[b, i] = jnp.where(split, 1, 0)
            slot = jnp.where(split, sp_b, 0)
            stack_ref[b, slot] = jnp.where(split, rc_b, stack_ref[b, slot])
            ra_ref[b, i] = rc_b
            scal_ref[0, b] = jnp.where(split, sp_b + 1, sp_b)
            scal_ref[1, b] = jnp.where(split, rc_b + 1, rc_b)
            is_leaf = jnp.logical_and(act, jnp.logical_not(split))
            la_ref[b, i] = lc_b
            scal_ref[2, b] = jnp.where(is_leaf, lc_b + 1, lc_b)
            scal_ref[3, b] = jnp.where(act, jnp.where(split, 1, 0), ps_b)
        return 0

    steps = sls_ref[0, 0]
    for b in range(1, B):
        steps = jnp.maximum(steps, sls_ref[0, b])
    jax.lax.fori_loop(0, steps, int_step, 0)

    wsum = wsum_ref[...]
    bias0 = bias0_ref[...]
    inv_d = 1.0 / D
    slv = slv_ref[...]
    col = jax.lax.broadcasted_iota(jnp.int32, (B, D), 1)

    def flt_step(i, carry):
        parent_prev, gate_prev, sfv, s2_prev, scores = carry
        left_prev = gate_prev * parent_prev

        ii = jnp.minimum(i, steps - 1)
        parts = []
        for b in range(B):
            parts.append(tree_ref[b, pl.ds(ga_ref[b, ii], 1), :])
        gath = jnp.concatenate(parts, axis=0)
        parent = jnp.where(sfv > 0.5, left_prev, gath)

        m = jnp.dot(parent, wl_ref[...], preferred_element_type=jnp.float32)
        s1 = jnp.sum(parent, axis=1, keepdims=True)
        s2 = jnp.sum(parent * parent, axis=1, keepdims=True)
        mu = s1 * inv_d
        var = s2 * inv_d - mu * mu
        istd = jax.lax.rsqrt(var + 1e-5)

        im1 = jnp.maximum(i - 1, 0)
        right_prev = parent_prev - left_prev
        spq = jnp.sum(parent_prev * left_prev, axis=1, keepdims=True)
        sq2 = jnp.sum(left_prev * left_prev, axis=1, keepdims=True)
        num = spq - sq2
        na = jnp.maximum(jnp.sqrt(sq2), 1e-8)
        nb = jnp.maximum(jnp.sqrt(s2_prev - 2.0 * spq + sq2), 1e-8)
        s = num / (na * nb)
        scores = scores + jnp.where((col == i - 1) & (slv > i - 1), s, 0.0)

        sf_parts = []
        for b in range(B):
            split_prev = jnp.logical_and(i > 0, sf_ref[b, im1] > 0)
            leaf_prev = jnp.logical_and(
                i > 0,
                jnp.logical_and(im1 < sls_ref[0, b], sf_ref[b, im1] == 0))
            ra_eff = jnp.where(split_prev, ra_ref[b, im1], ML - 1)
            la_eff = jnp.where(leaf_prev, la_ref[b, im1], ML)
            tree_ref[b, pl.ds(ra_eff, 1), :] = right_prev[b:b + 1, :]
            lscr_ref[b, pl.ds(la_eff, 1), :] = parent_prev[b:b + 1, :]

            sf_parts.append(jnp.full((1, 1),
                                     jnp.where(sf_ref[b, ii] > 0, 1.0, 0.0),
                                     jnp.float32))
        sfv_next = jnp.concatenate(sf_parts, axis=0)

        gate = jax.nn.sigmoid(istd * (m - mu * wsum) + bias0)
        return parent, gate, sfv_next, s2, scores

    scores = jax.lax.fori_loop(
        0, steps + 1, flt_step,
        (x_ref[...], jnp.ones((B, D), jnp.float32),
         jnp.ones((B, 1), jnp.float32), jnp.zeros((B, 1), jnp.float32),
         jnp.zeros((B, D), jnp.float32)),
    )[4]
    sc_ref[...] = scores
    leaf_ref[...] = lscr_ref[:, :ML, :]


def kernel(input_, features, length, label, ln_weight, ln_bias, lin_weight, lin_bias):
    del features
    length = length.astype(jnp.int32)
    label = label.astype(jnp.int32)
    sl = 2 * length - 1

    leaf, scores = pl.pallas_call(
        _splitnet_kernel,
        out_shape=[
            jax.ShapeDtypeStruct((B, ML, D), jnp.float32),
            jax.ShapeDtypeStruct((B, D), jnp.float32),
        ],
        in_specs=[
            pl.BlockSpec(memory_space=pltpu.VMEM),
            pl.BlockSpec(memory_space=pltpu.VMEM),
            pl.BlockSpec(memory_space=pltpu.VMEM),
            pl.BlockSpec(memory_space=pltpu.VMEM),
            pl.BlockSpec(memory_space=pltpu.VMEM),
            pl.BlockSpec(memory_space=pltpu.SMEM),
            pl.BlockSpec(memory_space=pltpu.SMEM),
        ],
        out_specs=[
            pl.BlockSpec(memory_space=pltpu.VMEM),
            pl.BlockSpec(memory_space=pltpu.VMEM),
        ],
        scratch_shapes=[
            pltpu.VMEM((B, ML, D), jnp.float32),
            pltpu.SMEM((B, D), jnp.int32),
            pltpu.SMEM((B, D), jnp.int32),
            pltpu.SMEM((B, D), jnp.int32),
            pltpu.SMEM((B, D), jnp.int32),
            pltpu.SMEM((B, D), jnp.int32),
            pltpu.SMEM((4, B), jnp.int32),
            pltpu.VMEM((B, ML + 8, D), jnp.float32),
        ],
    )(
        input_,
        ln_weight[:, None] * lin_weight.T,
        (ln_weight @ lin_weight.T).reshape(1, D),
        (ln_bias @ lin_weight.T + lin_bias).reshape(1, D),
        sl.reshape(B, 1),
        sl.reshape(1, B),
        label[:, :T],
    )
    return leaf, scores[:, :T]


# speedup vs baseline: 1.9491x; 1.1134x over previous
import jax
import jax.numpy as jnp
from jax.experimental import pallas as pl
from jax.experimental.pallas import tpu as pltpu

B = 8
D = 512
ML = 256
T = 2 * ML - 1


def _splitnet_kernel(x_ref, wl_ref, wsum_ref, bias0_ref, slv_ref,
                     sls_ref, lab_ref,
                     leaf_ref, sc_ref,
                     tree_ref, stack_ref, ga_ref, ra_ref, la_ref, sf_ref,
                     scal_ref, lscr_ref):
    lscr_ref[...] = jnp.zeros((B, ML + 8, D), jnp.float32)
    tree_ref[...] = jnp.zeros((B, ML, D), jnp.float32)

    for b in range(B):
        scal_ref[0, b] = 0
        scal_ref[1, b] = 0
        scal_ref[2, b] = 0
        scal_ref[3, b] = 1

    def int_step(i):
        for b in range(B):
            act = i < sls_ref[0, b]
            sp_b = scal_ref[0, b]
            rc_b = scal_ref[1, b]
            lc_b = scal_ref[2, b]
            ps_b = scal_ref[3, b]
            need_pop = jnp.logical_and(act, jnp.logical_and(ps_b == 0, sp_b > 0))
            ga_ref[b, i] = jnp.where(need_pop, stack_ref[b, jnp.where(need_pop, sp_b - 1, 0)], 0)
            sp_b = jnp.where(need_pop, sp_b - 1, sp_b)
            split = jnp.logical_and(act, lab_ref[b, i] > 0)
            sf_ref[b, i] = jnp.where(split, 1, 0)
            slot = jnp.where(split, sp_b, 0)
            stack_ref[b, slot] = jnp.where(split, rc_b, stack_ref[b, slot])
            ra_ref[b, i] = rc_b
            scal_ref[0, b] = jnp.where(split, sp_b + 1, sp_b)
            scal_ref[1, b] = jnp.where(split, rc_b + 1, rc_b)
            is_leaf = jnp.logical_and(act, jnp.logical_not(split))
            la_ref[b, i] = lc_b
            scal_ref[2, b] = jnp.where(is_leaf, lc_b + 1, lc_b)
            scal_ref[3, b] = jnp.where(act, jnp.where(split, 1, 0), ps_b)

    steps = sls_ref[0, 0]
    for b in range(1, B):
        steps = jnp.maximum(steps, sls_ref[0, b])
    LA = 2
    for j in range(LA):
        int_step(jnp.int32(j))

    wsum = wsum_ref[...]
    bias0 = bias0_ref[...]
    inv_d = 1.0 / D
    slv = slv_ref[...]
    col = jax.lax.broadcasted_iota(jnp.int32, (B, D), 1)

    def flt_step(i, carry):
        parent_prev, gate_prev, sfv, s2_prev, scores = carry
        left_prev = gate_prev * parent_prev

        int_step(i + LA)

        ii = jnp.minimum(i, steps - 1)
        parts = []
        for b in range(B):
            parts.append(tree_ref[b, pl.ds(ga_ref[b, ii], 1), :])
        gath = jnp.concatenate(parts, axis=0)
        parent = jnp.where(sfv > 0.5, left_prev, gath)

        m = jnp.dot(parent, wl_ref[...], preferred_element_type=jnp.float32)
        s1 = jnp.sum(parent, axis=1, keepdims=True)
        s2 = jnp.sum(parent * parent, axis=1, keepdims=True)
        mu = s1 * inv_d
        var = s2 * inv_d - mu * mu
        istd = jax.lax.rsqrt(var + 1e-5)

        im1 = jnp.maximum(i - 1, 0)
        right_prev = parent_prev - left_prev
        spq = jnp.sum(parent_prev * left_prev, axis=1, keepdims=True)
        sq2 = jnp.sum(left_prev * left_prev, axis=1, keepdims=True)
        num = spq - sq2
        na = jnp.maximum(jnp.sqrt(sq2), 1e-8)
        nb = jnp.maximum(jnp.sqrt(s2_prev - 2.0 * spq + sq2), 1e-8)
        s = num / (na * nb)
        scores = scores + jnp.where((col == i - 1) & (slv > i - 1), s, 0.0)

        sf_parts = []
        for b in range(B):
            split_prev = jnp.logical_and(i > 0, sf_ref[b, im1] > 0)
            leaf_prev = jnp.logical_and(
                i > 0,
                jnp.logical_and(im1 < sls_ref[0, b], sf_ref[b, im1] == 0))
            ra_eff = jnp.where(split_prev, ra_ref[b, im1], ML - 1)
            la_eff = jnp.where(leaf_prev, la_ref[b, im1], ML)
            tree_ref[b, pl.ds(ra_eff, 1), :] = right_prev[b:b + 1, :]
            lscr_ref[b, pl.ds(la_eff, 1), :] = parent_prev[b:b + 1, :]

            sf_parts.append(jnp.full((1, 1),
                                     jnp.where(sf_ref[b, ii] > 0, 1.0, 0.0),
                                     jnp.float32))
        sfv_next = jnp.concatenate(sf_parts, axis=0)

        gate = jax.nn.sigmoid(istd * (m - mu * wsum) + bias0)
        return parent, gate, sfv_next, s2, scores

    scores = jax.lax.fori_loop(
        0, steps + 1, flt_step,
        (x_ref[...], jnp.ones((B, D), jnp.float32),
         jnp.ones((B, 1), jnp.float32), jnp.zeros((B, 1), jnp.float32),
         jnp.zeros((B, D), jnp.float32)),
    )[4]
    sc_ref[...] = scores
    leaf_ref[...] = lscr_ref[:, :ML, :]


def kernel(input_, features, length, label, ln_weight, ln_bias, lin_weight, lin_bias):
    del features
    length = length.astype(jnp.int32)
    label = label.astype(jnp.int32)
    sl = 2 * length - 1

    leaf, scores = pl.pallas_call(
        _splitnet_kernel,
        out_shape=[
            jax.ShapeDtypeStruct((B, ML, D), jnp.float32),
            jax.ShapeDtypeStruct((B, D), jnp.float32),
        ],
        in_specs=[
            pl.BlockSpec(memory_space=pltpu.VMEM),
            pl.BlockSpec(memory_space=pltpu.VMEM),
            pl.BlockSpec(memory_space=pltpu.VMEM),
            pl.BlockSpec(memory_space=pltpu.VMEM),
            pl.BlockSpec(memory_space=pltpu.VMEM),
            pl.BlockSpec(memory_space=pltpu.SMEM),
            pl.BlockSpec(memory_space=pltpu.SMEM),
        ],
        out_specs=[
            pl.BlockSpec(memory_space=pltpu.VMEM),
            pl.BlockSpec(memory_space=pltpu.VMEM),
        ],
        scratch_shapes=[
            pltpu.VMEM((B, ML, D), jnp.float32),
            pltpu.SMEM((B, D), jnp.int32),
            pltpu.SMEM((B, D + 8), jnp.int32),
            pltpu.SMEM((B, D + 8), jnp.int32),
            pltpu.SMEM((B, D + 8), jnp.int32),
            pltpu.SMEM((B, D + 8), jnp.int32),
            pltpu.SMEM((4, B), jnp.int32),
            pltpu.VMEM((B, ML + 8, D), jnp.float32),
        ],
    )(
        input_,
        ln_weight[:, None] * lin_weight.T,
        (ln_weight @ lin_weight.T).reshape(1, D),
        (ln_bias @ lin_weight.T + lin_bias).reshape(1, D),
        sl.reshape(B, 1),
        sl.reshape(1, B),
        jnp.pad(label[:, :T], ((0, 0), (0, D + 8 - T))),
    )
    return leaf, scores[:, :T]


# speedup vs baseline: 1.9956x; 1.0239x over previous
import jax
import jax.numpy as jnp
from jax.experimental import pallas as pl
from jax.experimental.pallas import tpu as pltpu

B = 8
D = 512
ML = 256
T = 2 * ML - 1


def _splitnet_kernel(x_ref, wl_ref, wsum_ref, bias0_ref, slv_ref,
                     sls_ref, lab_ref,
                     leaf_ref, sc_ref,
                     tree_ref, stack_ref, ga_ref, ra_ref, la_ref, sf_ref,
                     scal_ref, lscr_ref):
    lscr_ref[...] = jnp.zeros((B, ML + 8, D), jnp.float32)
    tree_ref[...] = jnp.zeros((B, ML, D), jnp.float32)

    for b in range(B):
        scal_ref[0, b] = 0
        scal_ref[1, b] = 0
        scal_ref[2, b] = 0
        scal_ref[3, b] = 1

    def int_step(i):
        for b in range(B):
            act = i < sls_ref[0, b]
            sp_b = scal_ref[0, b]
            rc_b = scal_ref[1, b]
            lc_b = scal_ref[2, b]
            ps_b = scal_ref[3, b]
            need_pop = jnp.logical_and(act, jnp.logical_and(ps_b == 0, sp_b > 0))
            ga_ref[b, i] = jnp.where(need_pop, stack_ref[b, jnp.where(need_pop, sp_b - 1, 0)], 0)
            sp_b = jnp.where(need_pop, sp_b - 1, sp_b)
            split = jnp.logical_and(act, lab_ref[b, i] > 0)
            sf_ref[b, i] = jnp.where(split, 1, 0)
            slot = jnp.where(split, sp_b, 0)
            stack_ref[b, slot] = jnp.where(split, rc_b, stack_ref[b, slot])
            ra_ref[b, i] = rc_b
            scal_ref[0, b] = jnp.where(split, sp_b + 1, sp_b)
            scal_ref[1, b] = jnp.where(split, rc_b + 1, rc_b)
            is_leaf = jnp.logical_and(act, jnp.logical_not(split))
            la_ref[b, i] = lc_b
            scal_ref[2, b] = jnp.where(is_leaf, lc_b + 1, lc_b)
            scal_ref[3, b] = jnp.where(act, jnp.where(split, 1, 0), ps_b)

    steps = sls_ref[0, 0]
    for b in range(1, B):
        steps = jnp.maximum(steps, sls_ref[0, b])
    LA = 4
    for j in range(LA):
        int_step(jnp.int32(j))

    wsum = wsum_ref[...]
    bias0 = bias0_ref[...]
    inv_d = 1.0 / D
    slv = slv_ref[...]
    col = jax.lax.broadcasted_iota(jnp.int32, (B, D), 1)

    def flt_step(i, carry):
        parent_prev, gate_prev, sfv, s2_prev, scores = carry
        left_prev = gate_prev * parent_prev

        int_step(i + LA)

        parts = []
        for b in range(B):
            parts.append(tree_ref[b, pl.ds(ga_ref[b, i], 1), :])
        gath = jnp.concatenate(parts, axis=0)
        parent = jnp.where(sfv > 0.5, left_prev, gath)

        m = jnp.dot(parent, wl_ref[...], preferred_element_type=jnp.float32)
        s1 = jnp.sum(parent, axis=1, keepdims=True)
        s2 = jnp.sum(parent * parent, axis=1, keepdims=True)
        mu = s1 * inv_d
        var = s2 * inv_d - mu * mu
        istd = jax.lax.rsqrt(var + 1e-5)

        im1 = jnp.maximum(i - 1, 0)
        right_prev = parent_prev - left_prev
        spq = jnp.sum(parent_prev * left_prev, axis=1, keepdims=True)
        sq2 = jnp.sum(left_prev * left_prev, axis=1, keepdims=True)
        num = spq - sq2
        na = jnp.maximum(jnp.sqrt(sq2), 1e-8)
        nb = jnp.maximum(jnp.sqrt(s2_prev - 2.0 * spq + sq2), 1e-8)
        s = num / (na * nb)
        scores = scores + jnp.where((col == i - 1) & (slv > i - 1), s, 0.0)

        for b in range(B):
            split_prev = jnp.logical_and(i > 0, sf_ref[b, im1] > 0)
            leaf_prev = jnp.logical_and(
                i > 0,
                jnp.logical_and(im1 < sls_ref[0, b], sf_ref[b, im1] == 0))
            ra_eff = jnp.where(split_prev, ra_ref[b, im1], ML - 1)
            la_eff = jnp.where(leaf_prev, la_ref[b, im1], ML)
            tree_ref[b, pl.ds(ra_eff, 1), :] = right_prev[b:b + 1, :]
            lscr_ref[b, pl.ds(la_eff, 1), :] = parent_prev[b:b + 1, :]

        sf_parts = []
        for b in range(B):
            sf_parts.append(jnp.full((1, 1),
                                     jnp.where(sf_ref[b, i] > 0, 1.0, 0.0),
                                     jnp.float32))
        sfv_next = jnp.concatenate(sf_parts, axis=0)

        gate = jax.nn.sigmoid(istd * (m - mu * wsum) + bias0)
        return parent, gate, sfv_next, s2, scores

    scores = jax.lax.fori_loop(
        0, steps + 1, flt_step,
        (x_ref[...], jnp.ones((B, D), jnp.float32),
         jnp.ones((B, 1), jnp.float32), jnp.zeros((B, 1), jnp.float32),
         jnp.zeros((B, D), jnp.float32)),
    )[4]
    sc_ref[...] = scores
    leaf_ref[...] = lscr_ref[:, :ML, :]


def kernel(input_, features, length, label, ln_weight, ln_bias, lin_weight, lin_bias):
    del features
    length = length.astype(jnp.int32)
    label = label.astype(jnp.int32)
    sl = 2 * length - 1

    leaf, scores = pl.pallas_call(
        _splitnet_kernel,
        out_shape=[
            jax.ShapeDtypeStruct((B, ML, D), jnp.float32),
            jax.ShapeDtypeStruct((B, D), jnp.float32),
        ],
        in_specs=[
            pl.BlockSpec(memory_space=pltpu.VMEM),
            pl.BlockSpec(memory_space=pltpu.VMEM),
            pl.BlockSpec(memory_space=pltpu.VMEM),
            pl.BlockSpec(memory_space=pltpu.VMEM),
            pl.BlockSpec(memory_space=pltpu.VMEM),
            pl.BlockSpec(memory_space=pltpu.SMEM),
            pl.BlockSpec(memory_space=pltpu.SMEM),
        ],
        out_specs=[
            pl.BlockSpec(memory_space=pltpu.VMEM),
            pl.BlockSpec(memory_space=pltpu.VMEM),
        ],
        scratch_shapes=[
            pltpu.VMEM((B, ML, D), jnp.float32),
            pltpu.SMEM((B, D), jnp.int32),
            pltpu.SMEM((B, D + 8), jnp.int32),
            pltpu.SMEM((B, D + 8), jnp.int32),
            pltpu.SMEM((B, D + 8), jnp.int32),
            pltpu.SMEM((B, D + 8), jnp.int32),
            pltpu.SMEM((4, B), jnp.int32),
            pltpu.VMEM((B, ML + 8, D), jnp.float32),
        ],
    )(
        input_,
        ln_weight[:, None] * lin_weight.T,
        (ln_weight @ lin_weight.T).reshape(1, D),
        (ln_bias @ lin_weight.T + lin_bias).reshape(1, D),
        sl.reshape(B, 1),
        sl.reshape(1, B),
        jnp.pad(label[:, :T], ((0, 0), (0, D + 8 - T))),
    )
    return leaf, scores[:, :T]


# speedup vs baseline: 1.9967x; 1.0006x over previous
import jax
import jax.numpy as jnp
from jax.experimental import pallas as pl
from jax.experimental.pallas import tpu as pltpu

B = 8
D = 512
ML = 256
T = 2 * ML - 1


def _splitnet_kernel(x_ref, wl_ref, wsum_ref, bias0_ref, slv_ref,
                     sls_ref, lab_ref,
                     leaf_ref, sc_ref,
                     comb_ref, stack_ref, ga_ref, ra_ref, la_ref, sf_ref,
                     scal_ref):
    comb_ref[...] = jnp.zeros((B, 2 * ML + 16, D), jnp.float32)

    for b in range(B):
        scal_ref[0, b] = 0
        scal_ref[1, b] = 0
        scal_ref[2, b] = 0
        scal_ref[3, b] = 1

    def int_step(i):
        for b in range(B):
            act = i < sls_ref[0, b]
            sp_b = scal_ref[0, b]
            rc_b = scal_ref[1, b]
            lc_b = scal_ref[2, b]
            ps_b = scal_ref[3, b]
            need_pop = jnp.logical_and(act, jnp.logical_and(ps_b == 0, sp_b > 0))
            ga_ref[b, i] = jnp.where(need_pop, stack_ref[b, jnp.where(need_pop, sp_b - 1, 0)], 0)
            sp_b = jnp.where(need_pop, sp_b - 1, sp_b)
            split = jnp.logical_and(act, lab_ref[b, i] > 0)
            sf_ref[b, i] = jnp.where(split, 1, 0)
            slot = jnp.where(split, sp_b, 0)
            stack_ref[b, slot] = jnp.where(split, rc_b, stack_ref[b, slot])
            ra_ref[b, i] = rc_b
            scal_ref[0, b] = jnp.where(split, sp_b + 1, sp_b)
            scal_ref[1, b] = jnp.where(split, rc_b + 1, rc_b)
            is_leaf = jnp.logical_and(act, jnp.logical_not(split))
            la_ref[b, i] = lc_b
            scal_ref[2, b] = jnp.where(is_leaf, lc_b + 1, lc_b)
            scal_ref[3, b] = jnp.where(act, jnp.where(split, 1, 0), ps_b)

    steps = sls_ref[0, 0]
    for b in range(1, B):
        steps = jnp.maximum(steps, sls_ref[0, b])
    LA = 4
    for j in range(LA):
        int_step(jnp.int32(j))

    wsum = wsum_ref[...]
    bias0 = bias0_ref[...]
    inv_d = 1.0 / D
    slv = slv_ref[...]
    col = jax.lax.broadcasted_iota(jnp.int32, (B, D), 1)

    def flt_step(i, carry):
        parent_prev, gate_prev, sfv, s2_prev, scores = carry
        left_prev = gate_prev * parent_prev

        int_step(i + LA)

        parts = []
        for b in range(B):
            parts.append(comb_ref[b, pl.ds(ga_ref[b, i], 1), :])
        gath = jnp.concatenate(parts, axis=0)
        parent = jnp.where(sfv > 0.5, left_prev, gath)

        m = jnp.dot(parent, wl_ref[...], preferred_element_type=jnp.float32)
        s1 = jnp.sum(parent, axis=1, keepdims=True)
        s2 = jnp.sum(parent * parent, axis=1, keepdims=True)
        mu = s1 * inv_d
        var = s2 * inv_d - mu * mu
        istd = jax.lax.rsqrt(var + 1e-5)

        im1 = jnp.maximum(i - 1, 0)
        right_prev = parent_prev - left_prev
        spq = jnp.sum(parent_prev * left_prev, axis=1, keepdims=True)
        sq2 = jnp.sum(left_prev * left_prev, axis=1, keepdims=True)
        num = spq - sq2
        na = jnp.maximum(jnp.sqrt(sq2), 1e-8)
        nb = jnp.maximum(jnp.sqrt(s2_prev - 2.0 * spq + sq2), 1e-8)
        s = num / (na * nb)
        scores = scores + jnp.where((col == i - 1) & (slv > i - 1), s, 0.0)

        stval = jnp.where(sfv > 0.5, right_prev, parent_prev)
        for b in range(B):
            split_prev = jnp.logical_and(i > 0, sf_ref[b, im1] > 0)
            leaf_prev = jnp.logical_and(
                i > 0,
                jnp.logical_and(im1 < sls_ref[0, b], sf_ref[b, im1] == 0))
            addr = jnp.where(split_prev, ra_ref[b, im1],
                             jnp.where(leaf_prev, ML + la_ref[b, im1],
                                       2 * ML + 15))
            comb_ref[b, pl.ds(addr, 1), :] = stval[b:b + 1, :]

        sf_parts = []
        for b in range(B):
            sf_parts.append(jnp.full((1, 1),
                                     jnp.where(sf_ref[b, i] > 0, 1.0, 0.0),
                                     jnp.float32))
        sfv_next = jnp.concatenate(sf_parts, axis=0)

        gate = jax.nn.sigmoid(istd * (m - mu * wsum) + bias0)
        return parent, gate, sfv_next, s2, scores

    scores = jax.lax.fori_loop(
        0, steps + 1, flt_step,
        (x_ref[...], jnp.ones((B, D), jnp.float32),
         jnp.ones((B, 1), jnp.float32), jnp.zeros((B, 1), jnp.float32),
         jnp.zeros((B, D), jnp.float32)),
    )[4]
    sc_ref[...] = scores
    leaf_ref[...] = comb_ref[:, ML:2 * ML, :]


def kernel(input_, features, length, label, ln_weight, ln_bias, lin_weight, lin_bias):
    del features
    length = length.astype(jnp.int32)
    label = label.astype(jnp.int32)
    sl = 2 * length - 1

    leaf, scores = pl.pallas_call(
        _splitnet_kernel,
        out_shape=[
            jax.ShapeDtypeStruct((B, ML, D), jnp.float32),
            jax.ShapeDtypeStruct((B, D), jnp.float32),
        ],
        in_specs=[
            pl.BlockSpec(memory_space=pltpu.VMEM),
            pl.BlockSpec(memory_space=pltpu.VMEM),
            pl.BlockSpec(memory_space=pltpu.VMEM),
            pl.BlockSpec(memory_space=pltpu.VMEM),
            pl.BlockSpec(memory_space=pltpu.VMEM),
            pl.BlockSpec(memory_space=pltpu.SMEM),
            pl.BlockSpec(memory_space=pltpu.SMEM),
        ],
        out_specs=[
            pl.BlockSpec(memory_space=pltpu.VMEM),
            pl.BlockSpec(memory_space=pltpu.VMEM),
        ],
        scratch_shapes=[
            pltpu.VMEM((B, 2 * ML + 16, D), jnp.float32),
            pltpu.SMEM((B, D), jnp.int32),
            pltpu.SMEM((B, D + 8), jnp.int32),
            pltpu.SMEM((B, D + 8), jnp.int32),
            pltpu.SMEM((B, D + 8), jnp.int32),
            pltpu.SMEM((B, D + 8), jnp.int32),
            pltpu.SMEM((4, B), jnp.int32),
        ],
    )(
        input_,
        ln_weight[:, None] * lin_weight.T,
        (ln_weight @ lin_weight.T).reshape(1, D),
        (ln_bias @ lin_weight.T + lin_bias).reshape(1, D),
        sl.reshape(B, 1),
        sl.reshape(1, B),
        jnp.pad(label[:, :T], ((0, 0), (0, D + 8 - T))),
    )
    return leaf, scores[:, :T]


# speedup vs baseline: 2.1427x; 1.0731x over previous
import jax
import jax.numpy as jnp
from jax.experimental import pallas as pl
from jax.experimental.pallas import tpu as pltpu

B = 8
D = 512
ML = 256
T = 2 * ML - 1


def _splitnet_kernel(x_ref, wl_ref, wsum_ref, bias0_ref, slv_ref,
                     sls_ref, lab_ref,
                     leaf_ref, sc_ref,
                     comb_ref, stack_ref, ga_ref, ra_ref, la_ref, sf_ref,
                     scal_ref):
    comb_ref[...] = jnp.zeros((B, 2 * ML + 16, D), jnp.float32)

    for b in range(B):
        scal_ref[0, b] = 0
        scal_ref[1, b] = 0
        scal_ref[2, b] = 0
        scal_ref[3, b] = 1

    def int_step(i):
        for b in range(B):
            act = i < sls_ref[0, b]
            sp_b = scal_ref[0, b]
            rc_b = scal_ref[1, b]
            lc_b = scal_ref[2, b]
            ps_b = scal_ref[3, b]
            need_pop = jnp.logical_and(act, jnp.logical_and(ps_b == 0, sp_b > 0))
            ga_ref[b, i] = jnp.where(need_pop, stack_ref[b, jnp.where(need_pop, sp_b - 1, 0)], 0)
            sp_b = jnp.where(need_pop, sp_b - 1, sp_b)
            split = jnp.logical_and(act, lab_ref[b, i] > 0)
            sf_ref[b, i] = jnp.where(split, 1, 0)
            slot = jnp.where(split, sp_b, 0)
            stack_ref[b, slot] = jnp.where(split, rc_b, stack_ref[b, slot])
            ra_ref[b, i] = rc_b
            scal_ref[0, b] = jnp.where(split, sp_b + 1, sp_b)
            scal_ref[1, b] = jnp.where(split, rc_b + 1, rc_b)
            is_leaf = jnp.logical_and(act, jnp.logical_not(split))
            la_ref[b, i] = lc_b
            scal_ref[2, b] = jnp.where(is_leaf, lc_b + 1, lc_b)
            scal_ref[3, b] = jnp.where(act, jnp.where(split, 1, 0), ps_b)

    steps = sls_ref[0, 0]
    for b in range(1, B):
        steps = jnp.maximum(steps, sls_ref[0, b])
    LA = 4
    for j in range(LA):
        int_step(jnp.int32(j))

    wsum = wsum_ref[...]
    bias0 = bias0_ref[...]
    inv_d = 1.0 / D
    slv = slv_ref[...]
    col = jax.lax.broadcasted_iota(jnp.int32, (B, D), 1)

    DUMP = 2 * ML + 15

    def flt_step(i, carry):
        parent_prev, gate_prev, sfv, s2_prev, scores, gas, sts = carry
        left_prev = gate_prev * parent_prev

        parts = []
        for b in range(B):
            parts.append(comb_ref[b, pl.ds(gas[b], 1), :])
        gath = jnp.concatenate(parts, axis=0)
        parent = jnp.where(sfv > 0.5, left_prev, gath)

        m = jnp.dot(parent, wl_ref[...], preferred_element_type=jnp.float32)
        s1 = jnp.sum(parent, axis=1, keepdims=True)
        s2 = jnp.sum(parent * parent, axis=1, keepdims=True)
        mu = s1 * inv_d
        var = s2 * inv_d - mu * mu
        istd = jax.lax.rsqrt(var + 1e-5)

        int_step(i + LA)

        right_prev = parent_prev - left_prev
        spq = jnp.sum(parent_prev * left_prev, axis=1, keepdims=True)
        sq2 = jnp.sum(left_prev * left_prev, axis=1, keepdims=True)
        num = spq - sq2
        na = jnp.maximum(jnp.sqrt(sq2), 1e-8)
        nb = jnp.maximum(jnp.sqrt(s2_prev - 2.0 * spq + sq2), 1e-8)
        s = num / (na * nb)
        scores = scores + jnp.where((col == i - 1) & (slv > i - 1), s, 0.0)

        stval = jnp.where(sfv > 0.5, right_prev, parent_prev)
        for b in range(B):
            comb_ref[b, pl.ds(sts[b], 1), :] = stval[b:b + 1, :]

        gas_n = []
        sts_n = []
        sf_parts = []
        for b in range(B):
            gas_n.append(ga_ref[b, i + 1])
            sf_b = sf_ref[b, i]
            split_n = sf_b > 0
            leaf_n = jnp.logical_and(i < sls_ref[0, b], sf_b == 0)
            sts_n.append(jnp.where(split_n, ra_ref[b, i],
                                   jnp.where(leaf_n, ML + la_ref[b, i], DUMP)))
            sf_parts.append(jnp.full((1, 1),
                                     jnp.where(split_n, 1.0, 0.0),
                                     jnp.float32))
        sfv_next = jnp.concatenate(sf_parts, axis=0)

        gate = jax.nn.sigmoid(istd * (m - mu * wsum) + bias0)
        return (parent, gate, sfv_next, s2, scores,
                tuple(gas_n), tuple(sts_n))

    scores = jax.lax.fori_loop(
        0, steps + 1, flt_step,
        (x_ref[...], jnp.ones((B, D), jnp.float32),
         jnp.ones((B, 1), jnp.float32), jnp.zeros((B, 1), jnp.float32),
         jnp.zeros((B, D), jnp.float32),
         tuple(jnp.int32(0) for _ in range(B)),
         tuple(jnp.int32(DUMP) for _ in range(B))),
    )[4]
    sc_ref[...] = scores
    leaf_ref[...] = comb_ref[:, ML:2 * ML, :]


def kernel(input_, features, length, label, ln_weight, ln_bias, lin_weight, lin_bias):
    del features
    length = length.astype(jnp.int32)
    label = label.astype(jnp.int32)
    sl = 2 * length - 1

    leaf, scores = pl.pallas_call(
        _splitnet_kernel,
        out_shape=[
            jax.ShapeDtypeStruct((B, ML, D), jnp.float32),
            jax.ShapeDtypeStruct((B, D), jnp.float32),
        ],
        in_specs=[
            pl.BlockSpec(memory_space=pltpu.VMEM),
            pl.BlockSpec(memory_space=pltpu.VMEM),
            pl.BlockSpec(memory_space=pltpu.VMEM),
            pl.BlockSpec(memory_space=pltpu.VMEM),
            pl.BlockSpec(memory_space=pltpu.VMEM),
            pl.BlockSpec(memory_space=pltpu.SMEM),
            pl.BlockSpec(memory_space=pltpu.SMEM),
        ],
        out_specs=[
            pl.BlockSpec(memory_space=pltpu.VMEM),
            pl.BlockSpec(memory_space=pltpu.VMEM),
        ],
        scratch_shapes=[
            pltpu.VMEM((B, 2 * ML + 16, D), jnp.float32),
            pltpu.SMEM((B, D), jnp.int32),
            pltpu.SMEM((B, D + 8), jnp.int32),
            pltpu.SMEM((B, D + 8), jnp.int32),
            pltpu.SMEM((B, D + 8), jnp.int32),
            pltpu.SMEM((B, D + 8), jnp.int32),
            pltpu.SMEM((4, B), jnp.int32),
        ],
    )(
        input_,
        ln_weight[:, None] * lin_weight.T,
        (ln_weight @ lin_weight.T).reshape(1, D),
        (ln_bias @ lin_weight.T + lin_bias).reshape(1, D),
        sl.reshape(B, 1),
        sl.reshape(1, B),
        jnp.pad(label[:, :T], ((0, 0), (0, D + 8 - T))),
    )
    return leaf, scores[:, :T]


# speedup vs baseline: 2.6921x; 1.2564x over previous
import jax
import jax.numpy as jnp
from jax.experimental import pallas as pl
from jax.experimental.pallas import tpu as pltpu

B = 8
D = 512
ML = 256
T = 2 * ML - 1


def _splitnet_kernel(x_ref, wl_ref, wsum_ref, bias0_ref, slv_ref,
                     sls_ref, lab_ref,
                     leaf_ref, sc_ref,
                     comb_ref, stack_ref, ga_ref, ra_ref, la_ref, sf_ref,
                     scal_ref):
    comb_ref[...] = jnp.zeros((B, 2 * ML + 16, D), jnp.float32)

    for b in range(B):
        scal_ref[0, b] = 0
        scal_ref[1, b] = 0
        scal_ref[2, b] = 0
        scal_ref[3, b] = 1

    def int_step(i):
        for b in range(B):
            act = i < sls_ref[0, b]
            sp_b = scal_ref[0, b]
            rc_b = scal_ref[1, b]
            lc_b = scal_ref[2, b]
            ps_b = scal_ref[3, b]
            need_pop = jnp.logical_and(act, jnp.logical_and(ps_b == 0, sp_b > 0))
            ga_ref[b, i] = jnp.where(need_pop, stack_ref[b, jnp.where(need_pop, sp_b - 1, 0)], 0)
            sp_b = jnp.where(need_pop, sp_b - 1, sp_b)
            split = jnp.logical_and(act, lab_ref[b, i] > 0)
            sf_ref[b, i] = jnp.where(split, 1, 0)
            slot = jnp.where(split, sp_b, 0)
            stack_ref[b, slot] = jnp.where(split, rc_b, stack_ref[b, slot])
            ra_ref[b, i] = rc_b
            scal_ref[0, b] = jnp.where(split, sp_b + 1, sp_b)
            scal_ref[1, b] = jnp.where(split, rc_b + 1, rc_b)
            is_leaf = jnp.logical_and(act, jnp.logical_not(split))
            la_ref[b, i] = lc_b
            scal_ref[2, b] = jnp.where(is_leaf, lc_b + 1, lc_b)
            scal_ref[3, b] = jnp.where(act, jnp.where(split, 1, 0), ps_b)

    steps = sls_ref[0, 0]
    for b in range(1, B):
        steps = jnp.maximum(steps, sls_ref[0, b])
    LA = 4
    for j in range(LA):
        int_step(jnp.int32(j))

    wsum = wsum_ref[...]
    bias0 = bias0_ref[...]
    inv_d = 1.0 / D
    slv = slv_ref[...]
    col = jax.lax.broadcasted_iota(jnp.int32, (B, D), 1)

    DUMP = 2 * ML + 15

    def flt_step(i, carry):
        parent_prev, gate_prev, sfv, s2_prev, scores, sts, gath = carry
        left_prev = gate_prev * parent_prev

        parent = jnp.where(sfv > 0.5, left_prev, gath)

        m = jnp.dot(parent, wl_ref[...], preferred_element_type=jnp.float32)
        s1 = jnp.sum(parent, axis=1, keepdims=True)
        s2 = jnp.sum(parent * parent, axis=1, keepdims=True)
        mu = s1 * inv_d
        var = s2 * inv_d - mu * mu
        istd = jax.lax.rsqrt(var + 1e-5)

        int_step(i + LA)

        right_prev = parent_prev - left_prev
        spq = jnp.sum(parent_prev * left_prev, axis=1, keepdims=True)
        sq2 = jnp.sum(left_prev * left_prev, axis=1, keepdims=True)
        num = spq - sq2
        na = jnp.maximum(jnp.sqrt(sq2), 1e-8)
        nb = jnp.maximum(jnp.sqrt(s2_prev - 2.0 * spq + sq2), 1e-8)
        s = num / (na * nb)
        scores = scores + jnp.where((col == i - 1) & (slv > i - 1), s, 0.0)

        stval = jnp.where(sfv > 0.5, right_prev, parent_prev)
        for b in range(B):
            comb_ref[b, pl.ds(sts[b], 1), :] = stval[b:b + 1, :]

        sts_n = []
        sf_parts = []
        parts = []
        for b in range(B):
            parts.append(comb_ref[b, pl.ds(ga_ref[b, i + 1], 1), :])
            sf_b = sf_ref[b, i]
            split_n = sf_b > 0
            leaf_n = jnp.logical_and(i < sls_ref[0, b], sf_b == 0)
            sts_n.append(jnp.where(split_n, ra_ref[b, i],
                                   jnp.where(leaf_n, ML + la_ref[b, i], DUMP)))
            sf_parts.append(jnp.full((1, 1),
                                     jnp.where(split_n, 1.0, 0.0),
                                     jnp.float32))
        sfv_next = jnp.concatenate(sf_parts, axis=0)
        gath_next = jnp.concatenate(parts, axis=0)

        gate = jax.nn.sigmoid(istd * (m - mu * wsum) + bias0)
        return (parent, gate, sfv_next, s2, scores,
                tuple(sts_n), gath_next)

    scores = jax.lax.fori_loop(
        0, steps + 1, flt_step,
        (x_ref[...], jnp.ones((B, D), jnp.float32),
         jnp.ones((B, 1), jnp.float32), jnp.zeros((B, 1), jnp.float32),
         jnp.zeros((B, D), jnp.float32),
         tuple(jnp.int32(DUMP) for _ in range(B)),
         jnp.zeros((B, D), jnp.float32)),
    )[4]
    sc_ref[...] = scores
    leaf_ref[...] = comb_ref[:, ML:2 * ML, :]


def kernel(input_, features, length, label, ln_weight, ln_bias, lin_weight, lin_bias):
    del features
    length = length.astype(jnp.int32)
    label = label.astype(jnp.int32)
    sl = 2 * length - 1

    leaf, scores = pl.pallas_call(
        _splitnet_kernel,
        out_shape=[
            jax.ShapeDtypeStruct((B, ML, D), jnp.float32),
            jax.ShapeDtypeStruct((B, D), jnp.float32),
        ],
        in_specs=[
            pl.BlockSpec(memory_space=pltpu.VMEM),
            pl.BlockSpec(memory_space=pltpu.VMEM),
            pl.BlockSpec(memory_space=pltpu.VMEM),
            pl.BlockSpec(memory_space=pltpu.VMEM),
            pl.BlockSpec(memory_space=pltpu.VMEM),
            pl.BlockSpec(memory_space=pltpu.SMEM),
            pl.BlockSpec(memory_space=pltpu.SMEM),
        ],
        out_specs=[
            pl.BlockSpec(memory_space=pltpu.VMEM),
            pl.BlockSpec(memory_space=pltpu.VMEM),
        ],
        scratch_shapes=[
            pltpu.VMEM((B, 2 * ML + 16, D), jnp.float32),
            pltpu.SMEM((B, D), jnp.int32),
            pltpu.SMEM((B, D + 8), jnp.int32),
            pltpu.SMEM((B, D + 8), jnp.int32),
            pltpu.SMEM((B, D + 8), jnp.int32),
            pltpu.SMEM((B, D + 8), jnp.int32),
            pltpu.SMEM((4, B), jnp.int32),
        ],
    )(
        input_,
        ln_weight[:, None] * lin_weight.T,
        (ln_weight @ lin_weight.T).reshape(1, D),
        (ln_bias @ lin_weight.T + lin_bias).reshape(1, D),
        sl.reshape(B, 1),
        sl.reshape(1, B),
        jnp.pad(label[:, :T], ((0, 0), (0, D + 8 - T))),
    )
    return leaf, scores[:, :T]
